# baseline (speedup 1.0000x reference)
import functools

import jax
import jax.numpy as jnp
from jax import lax
from jax.experimental import pallas as pl
from jax.experimental.pallas import tpu as pltpu
from jax.experimental.pallas import tpu_sc as plsc

_V = 1000000
_B = 4096
_D = 64
_H = 20
_L = 16
_GIDX = _H * _L

_info = plsc.get_sparse_core_info()
_NC = _info.num_cores
_NS = _info.num_subcores
_NW = _NC * _NS
_NGROUP = _B // _L
_GPW = _NGROUP // _NW


_R = 128
_RP = 2 * _R
_NFULL = 3904
_TAIL = _V - _NFULL * _RP
_K1_ITERS = _NFULL // _NW


@functools.partial(
    pl.kernel,
    mesh=plsc.VectorSubcoreMesh(core_axis_name="c", subcore_axis_name="s"),
    out_type=jax.ShapeDtypeStruct((_V * _D,), jnp.float32),
    compiler_params=pltpu.CompilerParams(
        needs_layout_passes=False, use_tc_tiling_on_sc=True),
    scratch_types=[
        pltpu.VMEM((2, _D, _R), jnp.float32),
        pltpu.VMEM((2, _D, _R), jnp.float32),
        pltpu.VMEM((_RP * _D,), jnp.float32),
        pltpu.VMEM((_RP * _D,), jnp.float32),
        pltpu.SemaphoreType.DMA,
        pltpu.SemaphoreType.DMA,
        pltpu.SemaphoreType.DMA,
        pltpu.SemaphoreType.DMA,
    ],
)
def _relayout_table(tt_hbm, tail_hbm, out_hbm, in0, in1, out0, out1,
                    si0, si1, so0, so1):
    wid = lax.axis_index("s") * _NC + lax.axis_index("c")
    lane = lax.iota(jnp.int32, _L)
    lane_d = lane * _D

    def in_copies(c, buf, sem):
        v0 = pl.multiple_of(c * _RP, 128)
        return [
            pltpu.make_async_copy(
                tt_hbm.at[:, pl.ds(v0 + h * _R, _R)], buf.at[h], sem)
            for h in range(2)
        ]

    def out_slice(c):
        return out_hbm.at[pl.ds(c * (_RP * _D), _RP * _D)]

    def start_in(c, buf, sem):
        for cp in in_copies(c, buf, sem):
            cp.start()

    def process(c, k, inb, outb, sem_in, sem_out):
        for cp in in_copies(c, inb, sem_in):
            cp.wait()
        cprev = c - 2 * _NW

        @pl.when(cprev >= 0)
        def _():
            pltpu.make_async_copy(outb, out_slice(cprev), sem_out).wait()

        zeros_i = jnp.zeros((_L,), jnp.int32)
        fconst = []
        for r in range(_L):
            dsel = (lane + r) & 15
            fconst.append((
                ((dsel >> 3) << 10) + ((dsel & 7) << 7) + lane,
                lane_d + dsel,
            ))

        @plsc.parallel_loop(0, (_RP // _L) * 4, unroll=2)
        def _(t):
            vb = t >> 2
            g = t & 3
            h = vb >> 3
            vbl = vb & 7
            lbase = g * 2048 + vbl * _L
            sbase = vb * (_L * _D) + 16 * g
            for r in range(_L):
                fidx, sidx = fconst[r]
                vals = plsc.load_gather(inb, [zeros_i + h, zeros_i,
                                              fidx + lbase])
                plsc.store_scatter(outb, [sidx + sbase], vals)
        pltpu.async_copy(outb, out_slice(c), sem_out)

    npipe = _K1_ITERS // 2
    start_in(wid, in0, si0)
    start_in(wid + _NW, in1, si1)

    def pipe_body(k, c):
        c0 = 2 * k * _NW + wid
        c1 = c0 + _NW
        process(c0, k, in0, out0, si0, so0)

        @pl.when(k + 1 < npipe)
        def _():
            start_in(c0 + 2 * _NW, in0, si0)
        process(c1, k, in1, out1, si1, so1)

        @pl.when(k + 1 < npipe)
        def _():
            start_in(c1 + 2 * _NW, in1, si1)
        return c

    lax.fori_loop(0, npipe, pipe_body, 0)

    clast = (_K1_ITERS - 2) * _NW + wid
    pltpu.make_async_copy(out0, out_slice(clast), so0).wait()
    pltpu.make_async_copy(out1, out_slice(clast + _NW), so1).wait()

    @pl.when(wid == 0)
    def _():
        pltpu.sync_copy(tail_hbm,
                        out_hbm.at[pl.ds(_NFULL * _RP * _D, _TAIL * _D)])


@functools.partial(
    pl.kernel,
    mesh=plsc.VectorSubcoreMesh(core_axis_name="c", subcore_axis_name="s"),
    out_type=jax.ShapeDtypeStruct((_B,), jnp.float32),
    compiler_params=pltpu.CompilerParams(
        needs_layout_passes=False, use_tc_tiling_on_sc=False),
    scratch_types=[
        pltpu.VMEM((2, _GIDX), jnp.int32),
        pltpu.VMEM((2, _GIDX), jnp.int32),
        pltpu.VMEM((2, _L), jnp.int32),
        pltpu.VMEM((2, _GIDX, _D), jnp.float32),
        pltpu.VMEM((2, _GIDX, _D), jnp.float32),
        pltpu.VMEM((2, _L, _D), jnp.float32),
        pltpu.VMEM((_L,), jnp.float32),
        pltpu.SemaphoreType.DMA,
        pltpu.SemaphoreType.DMA,
        pltpu.SemaphoreType.DMA,
        pltpu.SemaphoreType.DMA,
        pltpu.SemaphoreType.DMA,
        pltpu.SemaphoreType.DMA,
    ],
)
def _distmult_sc(s_hbm, o_hbm, p_hbm, table_hbm, out_hbm,
                 s_idx, o_idx, p_idx, s_rows, o_rows, p_rows,
                 out_buf, ss0, so0, sp0, ss1, so1, sp1):
    wid = lax.axis_index("s") * _NC + lax.axis_index("c")
    zero = jnp.zeros((_L,), jnp.float32)
    lane = lax.iota(jnp.int32, _L)
    sems = [(ss0, so0, sp0), (ss1, so1, sp1)]

    def gathers(j, b):
        g = wid * _GPW + j
        sem_s, sem_o, sem_p = sems[b]
        return (
            pltpu.make_async_copy(table_hbm.at[s_idx.at[b]], s_rows.at[b],
                                  sem_s),
            pltpu.make_async_copy(table_hbm.at[o_idx.at[b]], o_rows.at[b],
                                  sem_o),
            pltpu.make_async_copy(table_hbm.at[p_idx.at[b]], p_rows.at[b],
                                  sem_p),
        )

    def prefetch(j, b):
        g = wid * _GPW + j
        pltpu.sync_copy(s_hbm.at[pl.ds(g * _GIDX, _GIDX)], s_idx.at[b])
        pltpu.sync_copy(o_hbm.at[pl.ds(g * _GIDX, _GIDX)], o_idx.at[b])
        pltpu.sync_copy(p_hbm.at[pl.ds(g * _L, _L)], p_idx.at[b])
        for cp in gathers(j, b):
            cp.start()

    def compute(j, b):
        g = wid * _GPW + j
        for cp in gathers(j, b):
            cp.wait()
        si = s_idx.at[b]
        oi = o_idx.at[b]
        sr = s_rows.at[b]
        orr = o_rows.at[b]
        pr = p_rows.at[b]

        def f_body(h, c):
            fs, fo = c
            fs = fs + jnp.where(si[pl.ds(h * _L, _L)] != 0, 1.0, 0.0)
            fo = fo + jnp.where(oi[pl.ds(h * _L, _L)] != 0, 1.0, 0.0)
            return fs, fo

        fs, fo = lax.fori_loop(0, _H, f_body, (zero, zero), unroll=4)
        scale = fs * fo

        def e_body(e, dots):
            def h_body(h, acc):
                a = list(acc)
                r = h * _L + e
                for gd in range(4):
                    sl = pl.ds(gd * _L, _L)
                    a[gd] = a[gd] + sr[r, sl]
                    a[4 + gd] = a[4 + gd] + orr[r, sl]
                return tuple(a)

            acc = lax.fori_loop(0, _H, h_body, (zero,) * 8, unroll=4)
            v = zero
            for gd in range(4):
                sl = pl.ds(gd * _L, _L)
                v = v + acc[gd] * pr[e, sl] * acc[4 + gd]
            return jnp.where(lane == e, jnp.sum(v), dots)

        dots = lax.fori_loop(0, _L, e_body, zero)

        x = scale * dots
        out_buf[...] = 1.0 / (1.0 + jnp.exp(-x))
        pltpu.sync_copy(out_buf, out_hbm.at[pl.ds(g * _L, _L)])

    prefetch(0, 0)

    def pipe(k, carry):
        j0 = 2 * k

        @pl.when(j0 + 1 < _GPW)
        def _():
            prefetch(j0 + 1, 1)
        compute(j0, 0)

        @pl.when(j0 + 2 < _GPW)
        def _():
            prefetch(j0 + 2, 0)
        compute(j0 + 1, 1)
        return carry

    lax.fori_loop(0, _GPW // 2, pipe, 0)


def kernel(s, o, p, table):
    s_flat = jnp.swapaxes(
        s.astype(jnp.int32).reshape(_NGROUP, _L, _H), 1, 2).reshape(-1)
    o_flat = jnp.swapaxes(
        o.astype(jnp.int32).reshape(_NGROUP, _L, _H), 1, 2).reshape(-1)
    p_flat = p.astype(jnp.int32).reshape(-1)
    tail = table[_NFULL * _RP:].reshape(-1)
    table_rm = _relayout_table(table.T, tail).reshape(_V, _D)
    return _distmult_sc(s_flat, o_flat, p_flat, table_rm)


# speedup vs baseline: 1.0042x; 1.0042x over previous
import functools

import jax
import jax.numpy as jnp
from jax import lax
from jax.experimental import pallas as pl
from jax.experimental.pallas import tpu as pltpu
from jax.experimental.pallas import tpu_sc as plsc

_V = 1000000
_B = 4096
_D = 64
_H = 20
_L = 16
_GIDX = _H * _L

_info = plsc.get_sparse_core_info()
_NC = _info.num_cores
_NS = _info.num_subcores
_NW = _NC * _NS
_NGROUP = _B // _L
_GPW = _NGROUP // _NW


_R = 128
_RP = 2 * _R
_NFULL = 3904
_TAIL = _V - _NFULL * _RP
_K1_ITERS = _NFULL // _NW


@functools.partial(
    pl.kernel,
    mesh=plsc.VectorSubcoreMesh(core_axis_name="c", subcore_axis_name="s"),
    out_type=jax.ShapeDtypeStruct((_V * _D,), jnp.float32),
    compiler_params=pltpu.CompilerParams(
        needs_layout_passes=False, use_tc_tiling_on_sc=True),
    scratch_types=[
        pltpu.VMEM((2, _D, _R), jnp.float32),
        pltpu.VMEM((2, _D, _R), jnp.float32),
        pltpu.VMEM((_RP * _D,), jnp.float32),
        pltpu.VMEM((_RP * _D,), jnp.float32),
        pltpu.SemaphoreType.DMA,
        pltpu.SemaphoreType.DMA,
        pltpu.SemaphoreType.DMA,
        pltpu.SemaphoreType.DMA,
    ],
)
def _relayout_table(tt_hbm, tail_hbm, out_hbm, in0, in1, out0, out1,
                    si0, si1, so0, so1):
    wid = lax.axis_index("s") * _NC + lax.axis_index("c")
    lane = lax.iota(jnp.int32, _L)
    lane_d = lane * _D

    def in_copies(c, buf, sem):
        v0 = pl.multiple_of(c * _RP, 128)
        return [
            pltpu.make_async_copy(
                tt_hbm.at[:, pl.ds(v0 + h * _R, _R)], buf.at[h], sem)
            for h in range(2)
        ]

    def out_slice(c):
        return out_hbm.at[pl.ds(c * (_RP * _D), _RP * _D)]

    def start_in(c, buf, sem):
        for cp in in_copies(c, buf, sem):
            cp.start()

    def process(c, k, inb, outb, sem_in, sem_out):
        for cp in in_copies(c, inb, sem_in):
            cp.wait()
        cprev = c - 2 * _NW

        @pl.when(cprev >= 0)
        def _():
            pltpu.make_async_copy(outb, out_slice(cprev), sem_out).wait()

        zeros_i = jnp.zeros((_L,), jnp.int32)
        fconst = []
        for r in range(_L):
            dsel = (lane + r) & 15
            fconst.append((
                ((dsel >> 3) << 10) + ((dsel & 7) << 7) + lane,
                lane_d + dsel,
            ))

        @plsc.parallel_loop(0, (_RP // _L) * 4, unroll=2)
        def _(t):
            vb = t >> 2
            g = t & 3
            h = vb >> 3
            vbl = vb & 7
            lbase = g * 2048 + vbl * _L
            sbase = vb * (_L * _D) + 16 * g
            for r in range(_L):
                fidx, sidx = fconst[r]
                vals = plsc.load_gather(inb, [zeros_i + h, zeros_i,
                                              fidx + lbase])
                plsc.store_scatter(outb, [sidx + sbase], vals)
        pltpu.async_copy(outb, out_slice(c), sem_out)

    npipe = _K1_ITERS // 2
    start_in(wid, in0, si0)
    start_in(wid + _NW, in1, si1)

    def pipe_body(k, c):
        c0 = 2 * k * _NW + wid
        c1 = c0 + _NW
        process(c0, k, in0, out0, si0, so0)

        @pl.when(k + 1 < npipe)
        def _():
            start_in(c0 + 2 * _NW, in0, si0)
        process(c1, k, in1, out1, si1, so1)

        @pl.when(k + 1 < npipe)
        def _():
            start_in(c1 + 2 * _NW, in1, si1)
        return c

    lax.fori_loop(0, npipe, pipe_body, 0)

    clast = (_K1_ITERS - 2) * _NW + wid
    pltpu.make_async_copy(out0, out_slice(clast), so0).wait()
    pltpu.make_async_copy(out1, out_slice(clast + _NW), so1).wait()

    @pl.when(wid == 0)
    def _():
        pltpu.sync_copy(tail_hbm,
                        out_hbm.at[pl.ds(_NFULL * _RP * _D, _TAIL * _D)])


@functools.partial(
    pl.kernel,
    mesh=plsc.VectorSubcoreMesh(core_axis_name="c", subcore_axis_name="s"),
    out_type=jax.ShapeDtypeStruct((_B,), jnp.float32),
    compiler_params=pltpu.CompilerParams(
        needs_layout_passes=False, use_tc_tiling_on_sc=False),
    scratch_types=[
        pltpu.VMEM((2, _GIDX), jnp.int32),
        pltpu.VMEM((2, _GIDX), jnp.int32),
        pltpu.VMEM((2, _L), jnp.int32),
        pltpu.VMEM((2, _GIDX, _D), jnp.float32),
        pltpu.VMEM((2, _GIDX, _D), jnp.float32),
        pltpu.VMEM((2, _L, _D), jnp.float32),
        pltpu.VMEM((_L,), jnp.float32),
        pltpu.SemaphoreType.DMA,
        pltpu.SemaphoreType.DMA,
        pltpu.SemaphoreType.DMA,
        pltpu.SemaphoreType.DMA,
        pltpu.SemaphoreType.DMA,
        pltpu.SemaphoreType.DMA,
    ],
)
def _distmult_sc(s_hbm, o_hbm, p_hbm, table_hbm, out_hbm,
                 s_idx, o_idx, p_idx, s_rows, o_rows, p_rows,
                 out_buf, ss0, so0, sp0, ss1, so1, sp1):
    wid = lax.axis_index("s") * _NC + lax.axis_index("c")
    zero = jnp.zeros((_L,), jnp.float32)
    lane = lax.iota(jnp.int32, _L)
    sems = [(ss0, so0, sp0), (ss1, so1, sp1)]

    def gathers(j, b):
        sem_s, sem_o, sem_p = sems[b]
        return (
            pltpu.make_async_copy(table_hbm.at[s_idx.at[b]], s_rows.at[b],
                                  sem_s),
            pltpu.make_async_copy(table_hbm.at[o_idx.at[b]], o_rows.at[b],
                                  sem_o),
            pltpu.make_async_copy(table_hbm.at[p_idx.at[b]], p_rows.at[b],
                                  sem_p),
        )

    def prefetch(j, b):
        g = wid * _GPW + j
        pltpu.sync_copy(s_hbm.at[pl.ds(g * _GIDX, _GIDX)], s_idx.at[b])
        pltpu.sync_copy(o_hbm.at[pl.ds(g * _GIDX, _GIDX)], o_idx.at[b])
        pltpu.sync_copy(p_hbm.at[pl.ds(g * _L, _L)], p_idx.at[b])
        for cp in gathers(j, b):
            cp.start()

    def compute(j, b):
        g = wid * _GPW + j
        for cp in gathers(j, b):
            cp.wait()
        si = s_idx.at[b]
        oi = o_idx.at[b]
        sr = s_rows.at[b]
        orr = o_rows.at[b]
        pr = p_rows.at[b]

        def f_body(h, c):
            fs, fo = c
            fs = fs + jnp.where(si[pl.ds(h * _L, _L)] != 0, 1.0, 0.0)
            fo = fo + jnp.where(oi[pl.ds(h * _L, _L)] != 0, 1.0, 0.0)
            return fs, fo

        fs, fo = lax.fori_loop(0, _H, f_body, (zero, zero), unroll=4)
        scale = fs * fo

        def e_body(e, dots):
            def h_body(h, acc):
                a = list(acc)
                r = h * _L + e
                for gd in range(4):
                    sl = pl.ds(gd * _L, _L)
                    a[gd] = a[gd] + sr[r, sl]
                    a[4 + gd] = a[4 + gd] + orr[r, sl]
                return tuple(a)

            acc = lax.fori_loop(0, _H, h_body, (zero,) * 8, unroll=4)
            v = zero
            for gd in range(4):
                sl = pl.ds(gd * _L, _L)
                v = v + acc[gd] * pr[e, sl] * acc[4 + gd]
            return jnp.where(lane == e, jnp.sum(v), dots)

        dots = lax.fori_loop(0, _L, e_body, zero)

        x = scale * dots
        out_buf[...] = 1.0 / (1.0 + jnp.exp(-x))
        pltpu.sync_copy(out_buf, out_hbm.at[pl.ds(g * _L, _L)])

    prefetch(0, 0)

    def pipe(k, carry):
        j0 = 2 * k

        @pl.when(j0 + 1 < _GPW)
        def _():
            prefetch(j0 + 1, 1)
        compute(j0, 0)

        @pl.when(j0 + 2 < _GPW)
        def _():
            prefetch(j0 + 2, 0)
        compute(j0 + 1, 1)
        return carry

    lax.fori_loop(0, _GPW // 2, pipe, 0)


def kernel(s, o, p, table):
    s_flat = jnp.swapaxes(
        s.astype(jnp.int32).reshape(_NGROUP, _L, _H), 1, 2).reshape(-1)
    o_flat = jnp.swapaxes(
        o.astype(jnp.int32).reshape(_NGROUP, _L, _H), 1, 2).reshape(-1)
    p_flat = p.astype(jnp.int32).reshape(-1)
    tail = table[_NFULL * _RP:].reshape(-1)
    table_rm = _relayout_table(table.T, tail).reshape(_V, _D)
    return _distmult_sc(s_flat, o_flat, p_flat, table_rm)
